# baseline (speedup 1.0000x reference)
import functools

import numpy as np
import jax
import jax.numpy as jnp
from jax import lax
from jax.experimental import pallas as pl
from jax.experimental.pallas import tpu as pltpu
from jax.experimental.pallas import tpu_sc as plsc

_B, _F0, _F1 = 1024, 10, 25
_N_NODES, _D = 100000, 128
_NC, _NS = 2, 16
_NW = _NC * _NS

_R2_PER_TILE = (_B * _F0 * _F1) // _NW
_CH2, _CW2 = 160, 50
_SEG2_PER_TILE = _R2_PER_TILE // _F1
_R1_PER_TILE = (_B * _F0) // _NW
_CH1, _CW1 = 8, 40
_SEG1_PER_TILE = _R1_PER_TILE // _F0
_R0_PER_TILE = _B // _NW

_SEG1 = np.asarray(
    ((np.arange(_B * _F0) // _F0) % (_NS * _SEG1_PER_TILE))
    .reshape(_NW, _CH1, _CW1), np.int32)

_f32 = jnp.float32


@functools.partial(
    pl.kernel,
    out_type=(
        jax.ShapeDtypeStruct((_B * _F0, _D), _f32),
        jax.ShapeDtypeStruct((_B * _F0, _D), _f32),
        jax.ShapeDtypeStruct((_B, _D), _f32),
        jax.ShapeDtypeStruct((_B, _D), _f32),
    ),
    mesh=plsc.VectorSubcoreMesh(
        core_axis_name="c", subcore_axis_name="s",
        num_cores=_NC, num_subcores=_NS),
    scratch_types=[
        pltpu.VMEM((_CH2, _CW2), jnp.int32),
        pltpu.VMEM((_CH1, _CW1), jnp.int32),
        pltpu.VMEM((_CH1, _CW1), jnp.int32),
        pltpu.VMEM((_R0_PER_TILE,), jnp.int32),
        pltpu.VMEM((_CW2, _D), _f32),
        pltpu.VMEM((_CW2, _D), _f32),
        pltpu.VMEM((_CW2, _D), _f32),
        pltpu.VMEM((_CW2, _D), _f32),
        pltpu.VMEM((_R0_PER_TILE, _D), _f32),
        pltpu.VMEM((_SEG2_PER_TILE, _D), _f32),
        pltpu.VMEM_SHARED((_NS * _SEG1_PER_TILE, _D), _f32),
        pltpu.SemaphoreType.DMA,
        pltpu.SemaphoreType.DMA,
        pltpu.SemaphoreType.DMA,
    ],
)
def _sc_gather(feat, idx2, idx1, seg1, idx0, zeros,
               sum2, g1, sum1, g0,
               idx2v, idx1v, seg1v, idx0v,
               buf_a, buf_b, buf_c, buf_d, buf0,
               sums2v, acc1, gsem, ssem, csem):
    c = lax.axis_index("c")
    s = lax.axis_index("s")
    t = c * _NS + s

    pltpu.sync_copy(idx2.at[t], idx2v)
    pltpu.sync_copy(idx1.at[t], idx1v)
    pltpu.sync_copy(seg1.at[t], seg1v)
    pltpu.sync_copy(idx0.at[t], idx0v)

    pltpu.async_copy(feat.at[idx0v], buf0, csem)

    pltpu.sync_copy(zeros.at[pl.ds(s * _SEG1_PER_TILE, _SEG1_PER_TILE)],
                    acc1.at[pl.ds(s * _SEG1_PER_TILE, _SEG1_PER_TILE)])

    bufs = (buf_a, buf_b, buf_c, buf_d)
    nbuf = len(bufs)
    for j in range(nbuf - 1):
        pltpu.async_copy(feat.at[idx2v.at[j]], bufs[j], gsem)

    def quad(m, carry):
        for b in range(nbuf):
            k = nbuf * m + b
            cur = bufs[b]
            prv = bufs[(b - 1) % nbuf]
            pltpu.make_async_copy(feat.at[idx2v.at[k]], cur, gsem).wait()
            @pl.when(k + nbuf - 1 < _CH2)
            def _():
                pltpu.async_copy(feat.at[idx2v.at[k + nbuf - 1]], prv, gsem)
            for seg in range(_CW2 // _F1):
                accs = [cur[seg * _F1, pl.ds(cc * 16, 16)] for cc in range(8)]
                for r in range(1, _F1):
                    for cc in range(8):
                        accs[cc] = accs[cc] + cur[seg * _F1 + r,
                                                  pl.ds(cc * 16, 16)]
                srow = (_CW2 // _F1) * k + seg
                for cc in range(8):
                    sums2v[srow, pl.ds(cc * 16, 16)] = accs[cc]
        return carry
    lax.fori_loop(0, _CH2 // nbuf, quad, 0)

    for w in range(_CH1 // nbuf):
        for j in range(nbuf):
            kk = w * nbuf + j
            pltpu.async_copy(feat.at[idx1v.at[kk]],
                             bufs[j].at[pl.ds(0, _CW1)], gsem)
        for j in range(nbuf):
            kk = w * nbuf + j
            dst = g1.at[pl.ds(t * _R1_PER_TILE + kk * _CW1, _CW1)]
            pltpu.make_async_copy(feat.at[idx1v.at[kk]],
                                  bufs[j].at[pl.ds(0, _CW1)], gsem).wait()
            pltpu.async_copy(bufs[j].at[pl.ds(0, _CW1)], dst, ssem)
            pltpu.sync_copy(bufs[j].at[pl.ds(0, _CW1)],
                            acc1.at[seg1v.at[kk]], add=True)
        for j in range(nbuf):
            kk = w * nbuf + j
            dst = g1.at[pl.ds(t * _R1_PER_TILE + kk * _CW1, _CW1)]
            pltpu.make_async_copy(bufs[j].at[pl.ds(0, _CW1)], dst,
                                  ssem).wait()

    pltpu.make_async_copy(feat.at[idx0v], buf0, csem).wait()
    pltpu.sync_copy(buf0, g0.at[pl.ds(t * _R0_PER_TILE, _R0_PER_TILE)])

    pltpu.sync_copy(sums2v,
                    sum2.at[pl.ds(t * _SEG2_PER_TILE, _SEG2_PER_TILE)])
    pltpu.sync_copy(acc1.at[pl.ds(s * _SEG1_PER_TILE, _SEG1_PER_TILE)],
                    sum1.at[pl.ds(t * _SEG1_PER_TILE, _SEG1_PER_TILE)])


def _tc_body(g0, g13, sum1, sum23, ws0, wn0, b0, ws1, wn1, b1, out):
    f32 = jnp.float32
    ws0v = ws0[...]
    b0v = b0[...]
    wn0v = wn0[...]
    x0 = jnp.maximum(
        jnp.dot(g0[...], ws0v, preferred_element_type=f32)
        + jnp.dot(sum1[...] * (1.0 / _F0), wn0v, preferred_element_type=f32)
        + b0v, 0.0)
    acc = jnp.zeros((_B, _D), f32)
    for r in range(_F0):
        x1r = jnp.maximum(
            jnp.dot(g13[:, r, :], ws0v, preferred_element_type=f32)
            + jnp.dot(sum23[:, r, :] * (1.0 / _F1), wn0v, preferred_element_type=f32)
            + b0v, 0.0)
        acc = acc + x1r
    out[...] = (jnp.dot(x0, ws1[...], preferred_element_type=f32)
                + jnp.dot(acc * (1.0 / _F0), wn1[...], preferred_element_type=f32)
                + b1[...])


def kernel(nodes, neigh1, neigh2, feat, W_self0, W_neigh0, b0,
           W_self1, W_neigh1, b1):
    idx2 = neigh2.astype(jnp.int32).reshape(_NW, _CH2, _CW2)
    idx1 = neigh1.astype(jnp.int32).reshape(_NW, _CH1, _CW1)
    idx0 = nodes.astype(jnp.int32).reshape(_NW, _R0_PER_TILE)
    zeros = jnp.zeros((_NS * _SEG1_PER_TILE, _D), _f32)

    sum2, g1, sum1, g0 = _sc_gather(
        feat, idx2, idx1, jnp.asarray(_SEG1), idx0, zeros)

    out = pl.pallas_call(
        _tc_body,
        out_shape=jax.ShapeDtypeStruct((_B, _D), _f32),
    )(g0, g1.reshape(_B, _F0, _D), sum1, sum2.reshape(_B, _F0, _D),
      W_self0, W_neigh0, b0.reshape(1, _D),
      W_self1, W_neigh1, b1.reshape(1, _D))
    return out


# speedup vs baseline: 1.2541x; 1.2541x over previous
import functools

import numpy as np
import jax
import jax.numpy as jnp
from jax import lax
from jax.experimental import pallas as pl
from jax.experimental.pallas import tpu as pltpu
from jax.experimental.pallas import tpu_sc as plsc

_B, _F0, _F1 = 1024, 10, 25
_N_NODES, _D = 100000, 128
_NC, _NS = 2, 16
_NW = _NC * _NS

_R2_PER_TILE = (_B * _F0 * _F1) // _NW
_CH2, _CW2 = 100, 80
_SEG2_PER_TILE = _R2_PER_TILE // _F1
_R1_PER_TILE = (_B * _F0) // _NW
_CH1, _CW1 = 4, 80
_SEG1_PER_TILE = _R1_PER_TILE // _F0
_R0_PER_TILE = _B // _NW

_SEG2 = np.asarray(
    ((np.arange(_B * _F0 * _F1) // _F1) % (_NS * _SEG2_PER_TILE))
    .reshape(_NW, _CH2, _CW2), np.int32)
_SEG1 = np.asarray(
    ((np.arange(_B * _F0) // _F0) % (_NS * _SEG1_PER_TILE))
    .reshape(_NW, _CH1, _CW1), np.int32)

_f32 = jnp.float32


@functools.partial(
    pl.kernel,
    out_type=(
        jax.ShapeDtypeStruct((_B * _F0, _D), _f32),
        jax.ShapeDtypeStruct((_B * _F0, _D), _f32),
        jax.ShapeDtypeStruct((_B, _D), _f32),
        jax.ShapeDtypeStruct((_B, _D), _f32),
    ),
    mesh=plsc.VectorSubcoreMesh(
        core_axis_name="c", subcore_axis_name="s",
        num_cores=_NC, num_subcores=_NS),
    scratch_types=[
        pltpu.VMEM((_CH2, _CW2), jnp.int32),
        pltpu.VMEM((_CH2, _CW2), jnp.int32),
        pltpu.VMEM((_CH1, _CW1), jnp.int32),
        pltpu.VMEM((_CH1, _CW1), jnp.int32),
        pltpu.VMEM((_R0_PER_TILE,), jnp.int32),
        pltpu.VMEM((_CW2, _D), _f32),
        pltpu.VMEM((_CW2, _D), _f32),
        pltpu.VMEM((_CW2, _D), _f32),
        pltpu.VMEM((_CW2, _D), _f32),
        pltpu.VMEM((_CW2, _D), _f32),
        pltpu.VMEM((_R0_PER_TILE, _D), _f32),
        pltpu.VMEM_SHARED((_NS * _SEG2_PER_TILE, _D), _f32),
        pltpu.VMEM_SHARED((_NS * _SEG1_PER_TILE, _D), _f32),
        pltpu.SemaphoreType.DMA,
        pltpu.SemaphoreType.DMA,
        pltpu.SemaphoreType.DMA,
    ],
)
def _sc_gather(feat, idx2, seg2, idx1, seg1, idx0, zeros,
               sum2, g1, sum1, g0,
               idx2v, seg2v, idx1v, seg1v, idx0v,
               buf_a, buf_b, buf_c, buf_d, buf_e, buf0,
               acc2, acc1, gsem, ssem, csem):
    c = lax.axis_index("c")
    s = lax.axis_index("s")
    t = c * _NS + s

    pltpu.sync_copy(idx2.at[t], idx2v)
    pltpu.sync_copy(seg2.at[t], seg2v)
    pltpu.sync_copy(idx1.at[t], idx1v)
    pltpu.sync_copy(seg1.at[t], seg1v)
    pltpu.sync_copy(idx0.at[t], idx0v)

    pltpu.async_copy(feat.at[idx0v], buf0, csem)

    pltpu.sync_copy(zeros.at[pl.ds(s * _SEG2_PER_TILE, _SEG2_PER_TILE)],
                    acc2.at[pl.ds(s * _SEG2_PER_TILE, _SEG2_PER_TILE)])
    pltpu.sync_copy(zeros.at[pl.ds(s * _SEG1_PER_TILE, _SEG1_PER_TILE)],
                    acc1.at[pl.ds(s * _SEG1_PER_TILE, _SEG1_PER_TILE)])

    bufs = (buf_a, buf_b, buf_c, buf_d, buf_e)
    nbuf = len(bufs)
    for j in range(nbuf - 1):
        pltpu.async_copy(feat.at[idx2v.at[j]], bufs[j], gsem)

    def quad(m, carry):
        for b in range(nbuf):
            k = nbuf * m + b
            cur = bufs[b]
            prv = bufs[(b - 1) % nbuf]
            pltpu.make_async_copy(feat.at[idx2v.at[k]], cur, gsem).wait()
            pltpu.async_copy(cur, acc2.at[seg2v.at[k]], ssem, add=True)
            @pl.when(k >= 1)
            def _():
                pltpu.make_async_copy(
                    prv, acc2.at[seg2v.at[k - 1]], ssem).wait()
            @pl.when(k + nbuf - 1 < _CH2)
            def _():
                pltpu.async_copy(feat.at[idx2v.at[k + nbuf - 1]], prv, gsem)
        return carry
    lax.fori_loop(0, _CH2 // nbuf, quad, 0)
    pltpu.make_async_copy(bufs[(_CH2 - 1) % nbuf],
                          acc2.at[seg2v.at[_CH2 - 1]], ssem).wait()

    for j in range(_CH1):
        pltpu.async_copy(feat.at[idx1v.at[j]], bufs[j], gsem)
    for j in range(_CH1):
        pltpu.make_async_copy(feat.at[idx1v.at[j]], bufs[j], gsem).wait()
        pltpu.async_copy(
            bufs[j], g1.at[pl.ds(t * _R1_PER_TILE + j * _CW1, _CW1)], ssem)
        pltpu.sync_copy(bufs[j], acc1.at[seg1v.at[j]], add=True)
    for j in range(_CH1):
        pltpu.make_async_copy(
            bufs[j], g1.at[pl.ds(t * _R1_PER_TILE + j * _CW1, _CW1)],
            ssem).wait()

    pltpu.make_async_copy(feat.at[idx0v], buf0, csem).wait()
    pltpu.sync_copy(buf0, g0.at[pl.ds(t * _R0_PER_TILE, _R0_PER_TILE)])

    pltpu.sync_copy(acc2.at[pl.ds(s * _SEG2_PER_TILE, _SEG2_PER_TILE)],
                    sum2.at[pl.ds(t * _SEG2_PER_TILE, _SEG2_PER_TILE)])
    pltpu.sync_copy(acc1.at[pl.ds(s * _SEG1_PER_TILE, _SEG1_PER_TILE)],
                    sum1.at[pl.ds(t * _SEG1_PER_TILE, _SEG1_PER_TILE)])


def _tc_body(g0, g13, sum1, sum23, ws0, wn0, b0, ws1, wn1, b1, out):
    f32 = jnp.float32
    ws0v = ws0[...]
    b0v = b0[...]
    wn0v = wn0[...]
    x0 = jnp.maximum(
        jnp.dot(g0[...], ws0v, preferred_element_type=f32)
        + jnp.dot(sum1[...] * (1.0 / _F0), wn0v, preferred_element_type=f32)
        + b0v, 0.0)
    acc = jnp.zeros((_B, _D), f32)
    for r in range(_F0):
        x1r = jnp.maximum(
            jnp.dot(g13[:, r, :], ws0v, preferred_element_type=f32)
            + jnp.dot(sum23[:, r, :] * (1.0 / _F1), wn0v, preferred_element_type=f32)
            + b0v, 0.0)
        acc = acc + x1r
    out[...] = (jnp.dot(x0, ws1[...], preferred_element_type=f32)
                + jnp.dot(acc * (1.0 / _F0), wn1[...], preferred_element_type=f32)
                + b1[...])


def kernel(nodes, neigh1, neigh2, feat, W_self0, W_neigh0, b0,
           W_self1, W_neigh1, b1):
    idx2 = neigh2.astype(jnp.int32).reshape(_NW, _CH2, _CW2)
    idx1 = neigh1.astype(jnp.int32).reshape(_NW, _CH1, _CW1)
    idx0 = nodes.astype(jnp.int32).reshape(_NW, _R0_PER_TILE)
    zeros = jnp.zeros((_NS * _SEG2_PER_TILE, _D), _f32)

    sum2, g1, sum1, g0 = _sc_gather(
        feat, idx2, jnp.asarray(_SEG2), idx1, jnp.asarray(_SEG1), idx0, zeros)

    out = pl.pallas_call(
        _tc_body,
        out_shape=jax.ShapeDtypeStruct((_B, _D), _f32),
    )(g0, g1.reshape(_B, _F0, _D), sum1, sum2.reshape(_B, _F0, _D),
      W_self0, W_neigh0, b0.reshape(1, _D),
      W_self1, W_neigh1, b1.reshape(1, _D))
    return out
